# Initial kernel scaffold; baseline (speedup 1.0000x reference)
import jax
import jax.numpy as jnp
from jax.experimental import pallas as pl


def kernel(x, edge_index, edge_attr, W1, b1, W2, b2, W3, b3):
    raise NotImplementedError("write your pallas kernel here")


# speedup vs baseline: 5.4509x; 5.4509x over previous
import functools

import jax
import jax.numpy as jnp
from jax import lax
from jax.experimental import pallas as pl
from jax.experimental.pallas import tpu as pltpu
from jax.experimental.pallas import tpu_sc as plsc

N = 10000
D = 128
E = 320000
LANES = 16
NC = 2
NS = 16
NW = NC * NS
EPW = E // NW
CH = 80
NCHUNK = EPW // CH
NPAD = 10240
RPS = NPAD // NS
ZB = 128
DSL = D // LANES

_mesh = plsc.VectorSubcoreMesh(
    core_axis_name="c", subcore_axis_name="s", num_cores=NC, num_subcores=NS)
_SC_PARAMS = pltpu.CompilerParams(use_tc_tiling_on_sc=False)


def _deg_body(zeros_hbm, col_hbm, ewx_hbm, out_hbm, col_v, ew_v, acc_sh):
  cid = lax.axis_index("c")
  sid = lax.axis_index("s")
  wid = sid * NC + cid

  pltpu.sync_copy(zeros_hbm, acc_sh.at[pl.ds(sid * RPS, RPS)])
  plsc.subcore_barrier()

  @pl.loop(0, NCHUNK)
  def _(ci):
    base = wid * EPW + ci * CH
    pltpu.sync_copy(col_hbm.at[pl.ds(base, CH)], col_v.at[0])
    pltpu.sync_copy(ewx_hbm.at[pl.ds(base, CH)], ew_v)
    pltpu.sync_copy(ew_v, acc_sh.at[col_v.at[0]], add=True)

  plsc.subcore_barrier()
  pltpu.sync_copy(acc_sh.at[pl.ds(sid * RPS, RPS)],
                  out_hbm.at[pl.ds(cid * NPAD + sid * RPS, RPS)])


_deg_call = pl.kernel(
    _deg_body,
    out_type=jax.ShapeDtypeStruct((NC * NPAD, LANES), jnp.float32),
    mesh=_mesh,
    scratch_types=[
        pltpu.VMEM((1, CH), jnp.int32),
        pltpu.VMEM((CH, LANES), jnp.float32),
        pltpu.VMEM_SHARED((NPAD, LANES), jnp.float32),
    ],
    compiler_params=_SC_PARAMS,
)


def _scatter_body(zerod_hbm, g_hbm, row_hbm, col_hbm, ewx_hbm, out_hbm,
                  row_v, col_v, ew_v, rows_v, acc_sh):
  cid = lax.axis_index("c")
  sid = lax.axis_index("s")
  wid = sid * NC + cid

  for k in range(RPS // ZB):
    pltpu.sync_copy(zerod_hbm, acc_sh.at[pl.ds(sid * RPS + k * ZB, ZB)])
  plsc.subcore_barrier()

  @pl.loop(0, NCHUNK)
  def _(ci):
    base = wid * EPW + ci * CH
    pltpu.sync_copy(row_hbm.at[pl.ds(base, CH)], row_v)
    pltpu.sync_copy(col_hbm.at[pl.ds(base, CH)], col_v.at[0])
    pltpu.sync_copy(ewx_hbm.at[pl.ds(base, CH)], ew_v)
    pltpu.sync_copy(g_hbm.at[row_v], rows_v)

    @pl.loop(0, CH)
    def _(e):
      w = ew_v[e]
      for j in range(DSL):
        sl = pl.ds(j * LANES, LANES)
        rows_v[e, sl] = rows_v[e, sl] * w

    pltpu.sync_copy(rows_v, acc_sh.at[col_v.at[0]], add=True)

  plsc.subcore_barrier()
  for k in range(RPS // ZB):
    r0 = sid * RPS + k * ZB
    pltpu.sync_copy(acc_sh.at[pl.ds(r0, ZB)],
                    out_hbm.at[pl.ds(cid * NPAD + r0, ZB)])


_scatter_call = pl.kernel(
    _scatter_body,
    out_type=jax.ShapeDtypeStruct((NC * NPAD, D), jnp.float32),
    mesh=_mesh,
    scratch_types=[
        pltpu.VMEM((CH,), jnp.int32),
        pltpu.VMEM((1, CH), jnp.int32),
        pltpu.VMEM((CH, LANES), jnp.float32),
        pltpu.VMEM((CH, D), jnp.float32),
        pltpu.VMEM_SHARED((NPAD, D), jnp.float32),
    ],
    compiler_params=_SC_PARAMS,
)


BN = 80
GRID = N // BN
PB = NPAD // BN
_DOT_DIMS = (((1,), (0,)), ((), ()))


def _mm1_kernel(d0_ref, d1_ref, x_ref, w_ref, dis_ref, g_ref):
  deg = d0_ref[:, :1] + d1_ref[:, :1] + 1.0
  dis = jnp.where(deg > 0, lax.rsqrt(deg), 0.0)
  h = lax.dot_general(x_ref[...], w_ref[...], _DOT_DIMS,
                      precision=lax.Precision.HIGHEST,
                      preferred_element_type=jnp.float32)
  dis_ref[...] = dis
  g_ref[...] = dis * h


_mm1_call = pl.pallas_call(
    _mm1_kernel,
    grid=(GRID,),
    in_specs=[
        pl.BlockSpec((BN, LANES), lambda i: (i, 0)),
        pl.BlockSpec((BN, LANES), lambda i: (i + PB, 0)),
        pl.BlockSpec((BN, D), lambda i: (i, 0)),
        pl.BlockSpec((D, D), lambda i: (0, 0)),
    ],
    out_specs=[
        pl.BlockSpec((BN, 1), lambda i: (i, 0)),
        pl.BlockSpec((BN, D), lambda i: (i, 0)),
    ],
    out_shape=[
        jax.ShapeDtypeStruct((N, 1), jnp.float32),
        jax.ShapeDtypeStruct((N, D), jnp.float32),
    ],
)


def _layer_kernel(p0_ref, p1_ref, g_ref, dis_ref, b_ref, w_ref, o_ref):
  dis = dis_ref[...]
  z = jnp.maximum(
      dis * (p0_ref[...] + p1_ref[...] + g_ref[...]) + b_ref[...], 0.0)
  h = lax.dot_general(z, w_ref[...], _DOT_DIMS,
                      precision=lax.Precision.HIGHEST,
                      preferred_element_type=jnp.float32)
  o_ref[...] = dis * h


_layer_call = pl.pallas_call(
    _layer_kernel,
    grid=(GRID,),
    in_specs=[
        pl.BlockSpec((BN, D), lambda i: (i, 0)),
        pl.BlockSpec((BN, D), lambda i: (i + PB, 0)),
        pl.BlockSpec((BN, D), lambda i: (i, 0)),
        pl.BlockSpec((BN, 1), lambda i: (i, 0)),
        pl.BlockSpec((1, D), lambda i: (0, 0)),
        pl.BlockSpec((D, D), lambda i: (0, 0)),
    ],
    out_specs=pl.BlockSpec((BN, D), lambda i: (i, 0)),
    out_shape=jax.ShapeDtypeStruct((N, D), jnp.float32),
)


def _final_kernel(p0_ref, p1_ref, g_ref, dis_ref, b_ref, o_ref):
  dis = dis_ref[...]
  o_ref[...] = jnp.maximum(
      dis * (p0_ref[...] + p1_ref[...] + g_ref[...]) + b_ref[...], 0.0)


_final_call = pl.pallas_call(
    _final_kernel,
    grid=(GRID,),
    in_specs=[
        pl.BlockSpec((BN, D), lambda i: (i, 0)),
        pl.BlockSpec((BN, D), lambda i: (i + PB, 0)),
        pl.BlockSpec((BN, D), lambda i: (i, 0)),
        pl.BlockSpec((BN, 1), lambda i: (i, 0)),
        pl.BlockSpec((1, D), lambda i: (0, 0)),
    ],
    out_specs=pl.BlockSpec((BN, D), lambda i: (i, 0)),
    out_shape=jax.ShapeDtypeStruct((N, D), jnp.float32),
)


def kernel(x, edge_index, edge_attr, W1, b1, W2, b2, W3, b3):
  row = edge_index[0]
  col = edge_index[1]
  ewx = jnp.broadcast_to(edge_attr[:, None], (E, LANES))
  b1r = b1.reshape(1, D)
  b2r = b2.reshape(1, D)
  b3r = b3.reshape(1, D)

  zs = jnp.zeros((RPS, LANES), jnp.float32)
  zd = jnp.zeros((ZB, D), jnp.float32)

  degp = _deg_call(zs, col, ewx)
  dis, g1 = _mm1_call(degp, degp, x, W1)
  p1 = _scatter_call(zd, g1, row, col, ewx)
  g2 = _layer_call(p1, p1, g1, dis, b1r, W2)
  p2 = _scatter_call(zd, g2, row, col, ewx)
  g3 = _layer_call(p2, p2, g2, dis, b2r, W3)
  p3 = _scatter_call(zd, g3, row, col, ewx)
  return _final_call(p3, p3, g3, dis, b3r)


# speedup vs baseline: 6.6283x; 1.2160x over previous
import functools

import jax
import jax.numpy as jnp
from jax import lax
from jax.experimental import pallas as pl
from jax.experimental.pallas import tpu as pltpu
from jax.experimental.pallas import tpu_sc as plsc

N = 10000
D = 128
E = 320000
LANES = 16
NC = 2
NS = 16
NW = NC * NS
EPW = E // NW
CH = 80
NCHUNK = EPW // CH
NPAD = 10240
RPS = NPAD // NS
ZB = 128
DSL = D // LANES

_mesh = plsc.VectorSubcoreMesh(
    core_axis_name="c", subcore_axis_name="s", num_cores=NC, num_subcores=NS)
_SC_PARAMS = pltpu.CompilerParams(use_tc_tiling_on_sc=False)


def _deg_body(zeros_hbm, col_hbm, ewx_hbm, out_hbm, col_v, ew_v, acc_sh):
  cid = lax.axis_index("c")
  sid = lax.axis_index("s")
  wid = sid * NC + cid

  pltpu.sync_copy(zeros_hbm, acc_sh.at[pl.ds(sid * RPS, RPS)])
  plsc.subcore_barrier()

  @pl.loop(0, NCHUNK)
  def _(ci):
    base = wid * EPW + ci * CH
    pltpu.sync_copy(col_hbm.at[pl.ds(base, CH)], col_v.at[0])
    pltpu.sync_copy(ewx_hbm.at[pl.ds(base, CH)], ew_v)
    pltpu.sync_copy(ew_v, acc_sh.at[col_v.at[0]], add=True)

  plsc.subcore_barrier()
  pltpu.sync_copy(acc_sh.at[pl.ds(sid * RPS, RPS)],
                  out_hbm.at[pl.ds(cid * NPAD + sid * RPS, RPS)])


_deg_call = pl.kernel(
    _deg_body,
    out_type=jax.ShapeDtypeStruct((NC * NPAD, LANES), jnp.float32),
    mesh=_mesh,
    scratch_types=[
        pltpu.VMEM((1, CH), jnp.int32),
        pltpu.VMEM((CH, LANES), jnp.float32),
        pltpu.VMEM_SHARED((NPAD, LANES), jnp.float32),
    ],
    compiler_params=_SC_PARAMS,
)


def _scatter_body(zerod_hbm, g_hbm, row_hbm, col_hbm, ewx_hbm, out_hbm,
                  row_v, col_v, ew_v, rows_v, acc_sh, sem_g):
  cid = lax.axis_index("c")
  sid = lax.axis_index("s")
  wid = sid * NC + cid

  for k in range(RPS // ZB):
    pltpu.sync_copy(zerod_hbm, acc_sh.at[pl.ds(sid * RPS + k * ZB, ZB)])
  plsc.subcore_barrier()

  def load_idx(ci, b):
    base = wid * EPW + ci * CH
    pltpu.sync_copy(row_hbm.at[pl.ds(base, CH)], row_v.at[b])
    pltpu.sync_copy(col_hbm.at[pl.ds(base, CH)], col_v.at[b])
    pltpu.sync_copy(ewx_hbm.at[pl.ds(base, CH)], ew_v.at[b])

  def start_gather(b):
    pltpu.async_copy(g_hbm.at[row_v.at[b]], rows_v.at[b], sem_g.at[b])

  def finish(b):
    pltpu.make_async_copy(g_hbm.at[row_v.at[b]], rows_v.at[b],
                          sem_g.at[b]).wait()

    @pl.loop(0, CH)
    def _(e):
      w = ew_v[b, e]
      for j in range(DSL):
        sl = pl.ds(j * LANES, LANES)
        rows_v[b, e, sl] = rows_v[b, e, sl] * w

    pltpu.sync_copy(rows_v.at[b], acc_sh.at[col_v.at[b]], add=True)

  load_idx(0, 0)
  start_gather(0)

  @pl.loop(0, (NCHUNK - 1) // 2)
  def _(g):
    c0 = 2 * g
    load_idx(c0 + 1, 1)
    start_gather(1)
    finish(0)
    load_idx(c0 + 2, 0)
    start_gather(0)
    finish(1)

  finish(0)

  plsc.subcore_barrier()
  for k in range(RPS // ZB):
    r0 = sid * RPS + k * ZB
    pltpu.sync_copy(acc_sh.at[pl.ds(r0, ZB)],
                    out_hbm.at[pl.ds(cid * NPAD + r0, ZB)])


_scatter_call = pl.kernel(
    _scatter_body,
    out_type=jax.ShapeDtypeStruct((NC * NPAD, D), jnp.float32),
    mesh=_mesh,
    scratch_types=[
        pltpu.VMEM((2, CH), jnp.int32),
        pltpu.VMEM((2, CH), jnp.int32),
        pltpu.VMEM((2, CH, LANES), jnp.float32),
        pltpu.VMEM((2, CH, D), jnp.float32),
        pltpu.VMEM_SHARED((NPAD, D), jnp.float32),
        pltpu.SemaphoreType.DMA((2,)),
    ],
    compiler_params=_SC_PARAMS,
)


BN = 80
GRID = N // BN
PB = NPAD // BN
_DOT_DIMS = (((1,), (0,)), ((), ()))


def _mm1_kernel(d0_ref, d1_ref, x_ref, w_ref, dis_ref, g_ref):
  deg = d0_ref[:, :1] + d1_ref[:, :1] + 1.0
  dis = jnp.where(deg > 0, lax.rsqrt(deg), 0.0)
  h = lax.dot_general(x_ref[...], w_ref[...], _DOT_DIMS,
                      precision=lax.Precision.HIGHEST,
                      preferred_element_type=jnp.float32)
  dis_ref[...] = dis
  g_ref[...] = dis * h


_mm1_call = pl.pallas_call(
    _mm1_kernel,
    grid=(GRID,),
    in_specs=[
        pl.BlockSpec((BN, LANES), lambda i: (i, 0)),
        pl.BlockSpec((BN, LANES), lambda i: (i + PB, 0)),
        pl.BlockSpec((BN, D), lambda i: (i, 0)),
        pl.BlockSpec((D, D), lambda i: (0, 0)),
    ],
    out_specs=[
        pl.BlockSpec((BN, 1), lambda i: (i, 0)),
        pl.BlockSpec((BN, D), lambda i: (i, 0)),
    ],
    out_shape=[
        jax.ShapeDtypeStruct((N, 1), jnp.float32),
        jax.ShapeDtypeStruct((N, D), jnp.float32),
    ],
)


def _layer_kernel(p0_ref, p1_ref, g_ref, dis_ref, b_ref, w_ref, o_ref):
  dis = dis_ref[...]
  z = jnp.maximum(
      dis * (p0_ref[...] + p1_ref[...] + g_ref[...]) + b_ref[...], 0.0)
  h = lax.dot_general(z, w_ref[...], _DOT_DIMS,
                      precision=lax.Precision.HIGHEST,
                      preferred_element_type=jnp.float32)
  o_ref[...] = dis * h


_layer_call = pl.pallas_call(
    _layer_kernel,
    grid=(GRID,),
    in_specs=[
        pl.BlockSpec((BN, D), lambda i: (i, 0)),
        pl.BlockSpec((BN, D), lambda i: (i + PB, 0)),
        pl.BlockSpec((BN, D), lambda i: (i, 0)),
        pl.BlockSpec((BN, 1), lambda i: (i, 0)),
        pl.BlockSpec((1, D), lambda i: (0, 0)),
        pl.BlockSpec((D, D), lambda i: (0, 0)),
    ],
    out_specs=pl.BlockSpec((BN, D), lambda i: (i, 0)),
    out_shape=jax.ShapeDtypeStruct((N, D), jnp.float32),
)


def _final_kernel(p0_ref, p1_ref, g_ref, dis_ref, b_ref, o_ref):
  dis = dis_ref[...]
  o_ref[...] = jnp.maximum(
      dis * (p0_ref[...] + p1_ref[...] + g_ref[...]) + b_ref[...], 0.0)


_final_call = pl.pallas_call(
    _final_kernel,
    grid=(GRID,),
    in_specs=[
        pl.BlockSpec((BN, D), lambda i: (i, 0)),
        pl.BlockSpec((BN, D), lambda i: (i + PB, 0)),
        pl.BlockSpec((BN, D), lambda i: (i, 0)),
        pl.BlockSpec((BN, 1), lambda i: (i, 0)),
        pl.BlockSpec((1, D), lambda i: (0, 0)),
    ],
    out_specs=pl.BlockSpec((BN, D), lambda i: (i, 0)),
    out_shape=jax.ShapeDtypeStruct((N, D), jnp.float32),
)


def kernel(x, edge_index, edge_attr, W1, b1, W2, b2, W3, b3):
  row = edge_index[0]
  col = edge_index[1]
  ewx = jnp.broadcast_to(edge_attr[:, None], (E, LANES))
  b1r = b1.reshape(1, D)
  b2r = b2.reshape(1, D)
  b3r = b3.reshape(1, D)

  zs = jnp.zeros((RPS, LANES), jnp.float32)
  zd = jnp.zeros((ZB, D), jnp.float32)

  degp = _deg_call(zs, col, ewx)
  dis, g1 = _mm1_call(degp, degp, x, W1)
  p1 = _scatter_call(zd, g1, row, col, ewx)
  g2 = _layer_call(p1, p1, g1, dis, b1r, W2)
  p2 = _scatter_call(zd, g2, row, col, ewx)
  g3 = _layer_call(p2, p2, g2, dis, b2r, W3)
  p3 = _scatter_call(zd, g3, row, col, ewx)
  return _final_call(p3, p3, g3, dis, b3r)


# speedup vs baseline: 6.7404x; 1.0169x over previous
import functools

import jax
import jax.numpy as jnp
from jax import lax
from jax.experimental import pallas as pl
from jax.experimental.pallas import tpu as pltpu
from jax.experimental.pallas import tpu_sc as plsc

N = 10000
D = 128
E = 320000
LANES = 16
NC = 2
NS = 16
NW = NC * NS
EPW = E // NW
CH = 80
NCHUNK = EPW // CH
NPAD = 10240
RPS = NPAD // NS
ZB = 128
DSL = D // LANES

_mesh = plsc.VectorSubcoreMesh(
    core_axis_name="c", subcore_axis_name="s", num_cores=NC, num_subcores=NS)
_SC_PARAMS = pltpu.CompilerParams(use_tc_tiling_on_sc=False)


def _deg_body(zeros_hbm, col_hbm, ewx_hbm, out_hbm, col_v, ew_v, acc_sh):
  cid = lax.axis_index("c")
  sid = lax.axis_index("s")
  wid = sid * NC + cid

  pltpu.sync_copy(zeros_hbm, acc_sh.at[pl.ds(sid * RPS, RPS)])
  plsc.subcore_barrier()

  @pl.loop(0, NCHUNK)
  def _(ci):
    base = wid * EPW + ci * CH
    pltpu.sync_copy(col_hbm.at[pl.ds(base, CH)], col_v.at[0])
    pltpu.sync_copy(ewx_hbm.at[pl.ds(base, CH)], ew_v)
    pltpu.sync_copy(ew_v, acc_sh.at[col_v.at[0]], add=True)

  plsc.subcore_barrier()
  pltpu.sync_copy(acc_sh.at[pl.ds(sid * RPS, RPS)],
                  out_hbm.at[pl.ds(cid * NPAD + sid * RPS, RPS)])


_deg_call = pl.kernel(
    _deg_body,
    out_type=jax.ShapeDtypeStruct((NC * NPAD, LANES), jnp.float32),
    mesh=_mesh,
    scratch_types=[
        pltpu.VMEM((1, CH), jnp.int32),
        pltpu.VMEM((CH, LANES), jnp.float32),
        pltpu.VMEM_SHARED((NPAD, LANES), jnp.float32),
    ],
    compiler_params=_SC_PARAMS,
)


def _scatter_body(zerod_hbm, g_hbm, row_hbm, col_hbm, ewx_hbm, out_hbm,
                  row_v, col_v, ew_v, rows_v, acc_sh, sem_g):
  cid = lax.axis_index("c")
  sid = lax.axis_index("s")
  wid = sid * NC + cid

  for k in range(RPS // ZB):
    pltpu.sync_copy(zerod_hbm, acc_sh.at[pl.ds(sid * RPS + k * ZB, ZB)])
  plsc.subcore_barrier()

  def load_idx(ci, b):
    base = wid * EPW + ci * CH
    pltpu.sync_copy(row_hbm.at[pl.ds(base, CH)], row_v.at[b])
    pltpu.sync_copy(col_hbm.at[pl.ds(base, CH)], col_v.at[b])
    pltpu.sync_copy(ewx_hbm.at[pl.ds(base, CH)], ew_v.at[b])

  def start_gather(b):
    pltpu.async_copy(g_hbm.at[row_v.at[b]], rows_v.at[b], sem_g.at[b])

  def finish(b):
    pltpu.make_async_copy(g_hbm.at[row_v.at[b]], rows_v.at[b],
                          sem_g.at[b]).wait()

    @pl.loop(0, CH, step=8)
    def _(eb):
      for de in range(8):
        e = eb + de
        w = ew_v[b, e]
        for j in range(DSL):
          sl = pl.ds(j * LANES, LANES)
          rows_v[b, e, sl] = rows_v[b, e, sl] * w

    pltpu.sync_copy(rows_v.at[b], acc_sh.at[col_v.at[b]], add=True)

  load_idx(0, 0)
  start_gather(0)

  @pl.loop(0, (NCHUNK - 1) // 2)
  def _(g):
    c0 = 2 * g
    load_idx(c0 + 1, 1)
    start_gather(1)
    finish(0)
    load_idx(c0 + 2, 0)
    start_gather(0)
    finish(1)

  finish(0)

  plsc.subcore_barrier()
  for k in range(RPS // ZB):
    r0 = sid * RPS + k * ZB
    pltpu.sync_copy(acc_sh.at[pl.ds(r0, ZB)],
                    out_hbm.at[pl.ds(cid * NPAD + r0, ZB)])


_scatter_call = pl.kernel(
    _scatter_body,
    out_type=jax.ShapeDtypeStruct((NC * NPAD, D), jnp.float32),
    mesh=_mesh,
    scratch_types=[
        pltpu.VMEM((2, CH), jnp.int32),
        pltpu.VMEM((2, CH), jnp.int32),
        pltpu.VMEM((2, CH, LANES), jnp.float32),
        pltpu.VMEM((2, CH, D), jnp.float32),
        pltpu.VMEM_SHARED((NPAD, D), jnp.float32),
        pltpu.SemaphoreType.DMA((2,)),
    ],
    compiler_params=_SC_PARAMS,
)


BN = 80
GRID = N // BN
PB = NPAD // BN
_DOT_DIMS = (((1,), (0,)), ((), ()))


def _mm1_kernel(d0_ref, d1_ref, x_ref, w_ref, dis_ref, g_ref):
  deg = d0_ref[:, :1] + d1_ref[:, :1] + 1.0
  dis = jnp.where(deg > 0, lax.rsqrt(deg), 0.0)
  h = lax.dot_general(x_ref[...], w_ref[...], _DOT_DIMS,
                      precision=lax.Precision.HIGHEST,
                      preferred_element_type=jnp.float32)
  dis_ref[...] = dis
  g_ref[...] = dis * h


_mm1_call = pl.pallas_call(
    _mm1_kernel,
    grid=(GRID,),
    in_specs=[
        pl.BlockSpec((BN, LANES), lambda i: (i, 0)),
        pl.BlockSpec((BN, LANES), lambda i: (i + PB, 0)),
        pl.BlockSpec((BN, D), lambda i: (i, 0)),
        pl.BlockSpec((D, D), lambda i: (0, 0)),
    ],
    out_specs=[
        pl.BlockSpec((BN, 1), lambda i: (i, 0)),
        pl.BlockSpec((BN, D), lambda i: (i, 0)),
    ],
    out_shape=[
        jax.ShapeDtypeStruct((N, 1), jnp.float32),
        jax.ShapeDtypeStruct((N, D), jnp.float32),
    ],
)


def _layer_kernel(p0_ref, p1_ref, g_ref, dis_ref, b_ref, w_ref, o_ref):
  dis = dis_ref[...]
  z = jnp.maximum(
      dis * (p0_ref[...] + p1_ref[...] + g_ref[...]) + b_ref[...], 0.0)
  h = lax.dot_general(z, w_ref[...], _DOT_DIMS,
                      precision=lax.Precision.HIGHEST,
                      preferred_element_type=jnp.float32)
  o_ref[...] = dis * h


_layer_call = pl.pallas_call(
    _layer_kernel,
    grid=(GRID,),
    in_specs=[
        pl.BlockSpec((BN, D), lambda i: (i, 0)),
        pl.BlockSpec((BN, D), lambda i: (i + PB, 0)),
        pl.BlockSpec((BN, D), lambda i: (i, 0)),
        pl.BlockSpec((BN, 1), lambda i: (i, 0)),
        pl.BlockSpec((1, D), lambda i: (0, 0)),
        pl.BlockSpec((D, D), lambda i: (0, 0)),
    ],
    out_specs=pl.BlockSpec((BN, D), lambda i: (i, 0)),
    out_shape=jax.ShapeDtypeStruct((N, D), jnp.float32),
)


def _final_kernel(p0_ref, p1_ref, g_ref, dis_ref, b_ref, o_ref):
  dis = dis_ref[...]
  o_ref[...] = jnp.maximum(
      dis * (p0_ref[...] + p1_ref[...] + g_ref[...]) + b_ref[...], 0.0)


_final_call = pl.pallas_call(
    _final_kernel,
    grid=(GRID,),
    in_specs=[
        pl.BlockSpec((BN, D), lambda i: (i, 0)),
        pl.BlockSpec((BN, D), lambda i: (i + PB, 0)),
        pl.BlockSpec((BN, D), lambda i: (i, 0)),
        pl.BlockSpec((BN, 1), lambda i: (i, 0)),
        pl.BlockSpec((1, D), lambda i: (0, 0)),
    ],
    out_specs=pl.BlockSpec((BN, D), lambda i: (i, 0)),
    out_shape=jax.ShapeDtypeStruct((N, D), jnp.float32),
)


def kernel(x, edge_index, edge_attr, W1, b1, W2, b2, W3, b3):
  row = edge_index[0]
  col = edge_index[1]
  ewx = jnp.broadcast_to(edge_attr[:, None], (E, LANES))
  b1r = b1.reshape(1, D)
  b2r = b2.reshape(1, D)
  b3r = b3.reshape(1, D)

  zs = jnp.zeros((RPS, LANES), jnp.float32)
  zd = jnp.zeros((ZB, D), jnp.float32)

  degp = _deg_call(zs, col, ewx)
  dis, g1 = _mm1_call(degp, degp, x, W1)
  p1 = _scatter_call(zd, g1, row, col, ewx)
  g2 = _layer_call(p1, p1, g1, dis, b1r, W2)
  p2 = _scatter_call(zd, g2, row, col, ewx)
  g3 = _layer_call(p2, p2, g2, dis, b2r, W3)
  p3 = _scatter_call(zd, g3, row, col, ewx)
  return _final_call(p3, p3, g3, dis, b3r)


# speedup vs baseline: 8.2002x; 1.2166x over previous
import functools

import jax
import jax.numpy as jnp
from jax import lax
from jax.experimental import pallas as pl
from jax.experimental.pallas import tpu as pltpu
from jax.experimental.pallas import tpu_sc as plsc

N = 10000
D = 128
E = 320000
LANES = 16
NC = 2
NS = 16
NW = NC * NS
EPW = E // NW
CH = 80
NCHUNK = EPW // CH
NPAD = 10240
RPS = NPAD // NS
ZB = 128
DSL = D // LANES

_mesh = plsc.VectorSubcoreMesh(
    core_axis_name="c", subcore_axis_name="s", num_cores=NC, num_subcores=NS)
_SC_PARAMS = pltpu.CompilerParams(use_tc_tiling_on_sc=False)


def _deg_body(zeros_hbm, col_hbm, ewx_hbm, out_hbm, col_v, ew_v, acc_sh):
  cid = lax.axis_index("c")
  sid = lax.axis_index("s")
  wid = sid * NC + cid

  pltpu.sync_copy(zeros_hbm, acc_sh.at[pl.ds(sid * RPS, RPS)])
  plsc.subcore_barrier()

  @pl.loop(0, NCHUNK)
  def _(ci):
    base = wid * EPW + ci * CH
    pltpu.sync_copy(col_hbm.at[pl.ds(base, CH)], col_v.at[0])
    pltpu.sync_copy(ewx_hbm.at[pl.ds(base, CH)], ew_v)
    pltpu.sync_copy(ew_v, acc_sh.at[col_v.at[0]], add=True)

  plsc.subcore_barrier()
  pltpu.sync_copy(acc_sh.at[pl.ds(sid * RPS, RPS)],
                  out_hbm.at[pl.ds(cid * NPAD + sid * RPS, RPS)])


_deg_call = pl.kernel(
    _deg_body,
    out_type=jax.ShapeDtypeStruct((NC * NPAD, LANES), jnp.float32),
    mesh=_mesh,
    scratch_types=[
        pltpu.VMEM((1, CH), jnp.int32),
        pltpu.VMEM((CH, LANES), jnp.float32),
        pltpu.VMEM_SHARED((NPAD, LANES), jnp.float32),
    ],
    compiler_params=_SC_PARAMS,
)


def _scatter_body(zerod_hbm, g_hbm, row_hbm, col_hbm, ewx_hbm, out_hbm,
                  row_v, col_v, ew_v, rows_v, acc_sh, sem_g, sem_i):
  cid = lax.axis_index("c")
  sid = lax.axis_index("s")
  wid = sid * NC + cid

  for k in range(RPS // ZB):
    pltpu.sync_copy(zerod_hbm, acc_sh.at[pl.ds(sid * RPS + k * ZB, ZB)])
  plsc.subcore_barrier()

  def load_idx(ci, b):
    base = wid * EPW + ci * CH
    pltpu.async_copy(row_hbm.at[pl.ds(base, CH)], row_v.at[b], sem_i.at[b])
    pltpu.async_copy(col_hbm.at[pl.ds(base, CH)], col_v.at[b], sem_i.at[b])
    pltpu.async_copy(ewx_hbm.at[pl.ds(base, CH)], ew_v.at[b], sem_i.at[b])
    pltpu.make_async_copy(row_hbm.at[pl.ds(base, CH)], row_v.at[b],
                          sem_i.at[b]).wait()
    pltpu.make_async_copy(col_hbm.at[pl.ds(base, CH)], col_v.at[b],
                          sem_i.at[b]).wait()
    pltpu.make_async_copy(ewx_hbm.at[pl.ds(base, CH)], ew_v.at[b],
                          sem_i.at[b]).wait()

  def start_gather(b):
    pltpu.async_copy(g_hbm.at[row_v.at[b]], rows_v.at[b], sem_g.at[b])

  def finish(b):
    pltpu.make_async_copy(g_hbm.at[row_v.at[b]], rows_v.at[b],
                          sem_g.at[b]).wait()

    @pl.loop(0, CH, step=8)
    def _(eb):
      for de in range(8):
        e = eb + de
        w = ew_v[b, e]
        for j in range(DSL):
          sl = pl.ds(j * LANES, LANES)
          rows_v[b, e, sl] = rows_v[b, e, sl] * w

    pltpu.sync_copy(rows_v.at[b], acc_sh.at[col_v.at[b]], add=True)

  load_idx(0, 0)
  start_gather(0)

  @pl.loop(0, (NCHUNK - 1) // 2)
  def _(g):
    c0 = 2 * g
    load_idx(c0 + 1, 1)
    start_gather(1)
    finish(0)
    load_idx(c0 + 2, 0)
    start_gather(0)
    finish(1)

  finish(0)

  plsc.subcore_barrier()
  for k in range(RPS // ZB):
    r0 = sid * RPS + k * ZB
    pltpu.sync_copy(acc_sh.at[pl.ds(r0, ZB)],
                    out_hbm.at[pl.ds(cid * NPAD + r0, ZB)])


_scatter_call = pl.kernel(
    _scatter_body,
    out_type=jax.ShapeDtypeStruct((NC * NPAD, D), jnp.float32),
    mesh=_mesh,
    scratch_types=[
        pltpu.VMEM((2, CH), jnp.int32),
        pltpu.VMEM((2, CH), jnp.int32),
        pltpu.VMEM((2, CH, LANES), jnp.float32),
        pltpu.VMEM((2, CH, D), jnp.float32),
        pltpu.VMEM_SHARED((NPAD, D), jnp.float32),
        pltpu.SemaphoreType.DMA((2,)),
        pltpu.SemaphoreType.DMA((2,)),
    ],
    compiler_params=_SC_PARAMS,
)


BN = 80
GRID = N // BN
PB = NPAD // BN
_DOT_DIMS = (((1,), (0,)), ((), ()))


def _mm1_kernel(d0_ref, d1_ref, x_ref, w_ref, dis_ref, g_ref):
  deg = d0_ref[:, :1] + d1_ref[:, :1] + 1.0
  dis = jnp.where(deg > 0, lax.rsqrt(deg), 0.0)
  h = lax.dot_general(x_ref[...], w_ref[...], _DOT_DIMS,
                      precision=lax.Precision.HIGHEST,
                      preferred_element_type=jnp.float32)
  dis_ref[...] = dis
  g_ref[...] = dis * h


_mm1_call = pl.pallas_call(
    _mm1_kernel,
    grid=(GRID,),
    in_specs=[
        pl.BlockSpec((BN, LANES), lambda i: (i, 0)),
        pl.BlockSpec((BN, LANES), lambda i: (i + PB, 0)),
        pl.BlockSpec((BN, D), lambda i: (i, 0)),
        pl.BlockSpec((D, D), lambda i: (0, 0)),
    ],
    out_specs=[
        pl.BlockSpec((BN, 1), lambda i: (i, 0)),
        pl.BlockSpec((BN, D), lambda i: (i, 0)),
    ],
    out_shape=[
        jax.ShapeDtypeStruct((N, 1), jnp.float32),
        jax.ShapeDtypeStruct((N, D), jnp.float32),
    ],
)


def _layer_kernel(p0_ref, p1_ref, g_ref, dis_ref, b_ref, w_ref, o_ref):
  dis = dis_ref[...]
  z = jnp.maximum(
      dis * (p0_ref[...] + p1_ref[...] + g_ref[...]) + b_ref[...], 0.0)
  h = lax.dot_general(z, w_ref[...], _DOT_DIMS,
                      precision=lax.Precision.HIGHEST,
                      preferred_element_type=jnp.float32)
  o_ref[...] = dis * h


_layer_call = pl.pallas_call(
    _layer_kernel,
    grid=(GRID,),
    in_specs=[
        pl.BlockSpec((BN, D), lambda i: (i, 0)),
        pl.BlockSpec((BN, D), lambda i: (i + PB, 0)),
        pl.BlockSpec((BN, D), lambda i: (i, 0)),
        pl.BlockSpec((BN, 1), lambda i: (i, 0)),
        pl.BlockSpec((1, D), lambda i: (0, 0)),
        pl.BlockSpec((D, D), lambda i: (0, 0)),
    ],
    out_specs=pl.BlockSpec((BN, D), lambda i: (i, 0)),
    out_shape=jax.ShapeDtypeStruct((N, D), jnp.float32),
)


def _final_kernel(p0_ref, p1_ref, g_ref, dis_ref, b_ref, o_ref):
  dis = dis_ref[...]
  o_ref[...] = jnp.maximum(
      dis * (p0_ref[...] + p1_ref[...] + g_ref[...]) + b_ref[...], 0.0)


_final_call = pl.pallas_call(
    _final_kernel,
    grid=(GRID,),
    in_specs=[
        pl.BlockSpec((BN, D), lambda i: (i, 0)),
        pl.BlockSpec((BN, D), lambda i: (i + PB, 0)),
        pl.BlockSpec((BN, D), lambda i: (i, 0)),
        pl.BlockSpec((BN, 1), lambda i: (i, 0)),
        pl.BlockSpec((1, D), lambda i: (0, 0)),
    ],
    out_specs=pl.BlockSpec((BN, D), lambda i: (i, 0)),
    out_shape=jax.ShapeDtypeStruct((N, D), jnp.float32),
)


def kernel(x, edge_index, edge_attr, W1, b1, W2, b2, W3, b3):
  row = edge_index[0]
  col = edge_index[1]
  ewx = jnp.broadcast_to(edge_attr[:, None], (E, LANES))
  b1r = b1.reshape(1, D)
  b2r = b2.reshape(1, D)
  b3r = b3.reshape(1, D)

  zs = jnp.zeros((RPS, LANES), jnp.float32)
  zd = jnp.zeros((ZB, D), jnp.float32)

  degp = _deg_call(zs, col, ewx)
  dis, g1 = _mm1_call(degp, degp, x, W1)
  p1 = _scatter_call(zd, g1, row, col, ewx)
  g2 = _layer_call(p1, p1, g1, dis, b1r, W2)
  p2 = _scatter_call(zd, g2, row, col, ewx)
  g3 = _layer_call(p2, p2, g2, dis, b2r, W3)
  p3 = _scatter_call(zd, g3, row, col, ewx)
  return _final_call(p3, p3, g3, dis, b3r)
